# Initial kernel scaffold; baseline (speedup 1.0000x reference)
import jax
import jax.numpy as jnp
from jax.experimental import pallas as pl


def kernel(x, edge_index, batch, W1, b1, g1, be1, W2, b2, g2, be2, Wf1, bf1, g3, be3, Wf2, bf2, Wf3, bf3):
    raise NotImplementedError("write your pallas kernel here")


# speedup vs baseline: 13.4196x; 13.4196x over previous
import functools

import jax
import jax.numpy as jnp
from jax import lax
from jax.experimental import pallas as pl
from jax.experimental.pallas import tpu as pltpu
from jax.experimental.pallas import tpu_sc as plsc

N = 10000
E = 160000
D = 256
G = 64

NC = 2
NS = 16
LANES = 16
NW = NC * NS
EPT = E // NW
CH = 128
NFULL = EPT // CH
TAIL = EPT - NFULL * CH
RPA = 624
REM = N - NS * RPA
ZCH = 104
EPS = 1e-5

_HIGH = lax.Precision.HIGHEST


def _mesh():
    return plsc.VectorSubcoreMesh(core_axis_name="c", subcore_axis_name="s",
                                  num_cores=NC, num_subcores=NS)


@functools.lru_cache(maxsize=None)
def _make_deg():
    @functools.partial(
        pl.kernel,
        out_type=jax.ShapeDtypeStruct((NC, N, LANES), jnp.float32),
        mesh=_mesh(),
        scratch_types=[
            pltpu.VMEM((CH,), jnp.int32),
            pltpu.VMEM((TAIL,), jnp.int32),
            pltpu.VMEM((CH, LANES), jnp.float32),
            pltpu.VMEM_SHARED((N, LANES), jnp.float32),
        ],
    )
    def deg_kernel(dst_hbm, out_hbm, idx_d, idx_dt, ones_v, table):
        c = lax.axis_index("c")
        s = lax.axis_index("s")
        wid = s * NC + c
        zeros16 = jnp.zeros((LANES,), jnp.float32)

        def zrow(r, carry):
            ones_v[r, :] = zeros16
            return carry

        lax.fori_loop(0, CH, zrow, None)
        for k in range(RPA // ZCH):
            pltpu.sync_copy(ones_v.at[pl.ds(0, ZCH)],
                            table.at[pl.ds(s * RPA + k * ZCH, ZCH)])

        @pl.when(s == 0)
        def _ztail():
            pltpu.sync_copy(ones_v.at[pl.ds(0, REM)],
                            table.at[pl.ds(NS * RPA, REM)])

        ones16 = jnp.ones((LANES,), jnp.float32)

        def orow(r, carry):
            ones_v[r, :] = ones16
            return carry

        lax.fori_loop(0, CH, orow, None)
        plsc.subcore_barrier()

        ebase = wid * EPT

        def chunk(ci, carry):
            off = ebase + ci * CH
            pltpu.sync_copy(dst_hbm.at[pl.ds(off, CH)], idx_d)
            pltpu.sync_copy(ones_v, table.at[idx_d], add=True)
            return carry

        lax.fori_loop(0, NFULL, chunk, None)
        pltpu.sync_copy(dst_hbm.at[pl.ds(ebase + NFULL * CH, TAIL)], idx_dt)
        pltpu.sync_copy(ones_v.at[pl.ds(0, TAIL)], table.at[idx_dt], add=True)

        plsc.subcore_barrier()
        pltpu.sync_copy(table.at[pl.ds(s * RPA, RPA)],
                        out_hbm.at[c, pl.ds(s * RPA, RPA)])

        @pl.when(s == 0)
        def _rtail():
            pltpu.sync_copy(table.at[pl.ds(NS * RPA, REM)],
                            out_hbm.at[c, pl.ds(NS * RPA, REM)])

    return deg_kernel


@functools.lru_cache(maxsize=None)
def _make_prop(F):
    @functools.partial(
        pl.kernel,
        out_type=jax.ShapeDtypeStruct((NC, N, F), jnp.float32),
        mesh=_mesh(),
        scratch_types=[
            pltpu.VMEM((CH,), jnp.int32),
            pltpu.VMEM((CH,), jnp.int32),
            pltpu.VMEM((TAIL,), jnp.int32),
            pltpu.VMEM((TAIL,), jnp.int32),
            pltpu.VMEM((CH, F), jnp.float32),
            pltpu.VMEM_SHARED((N, F), jnp.float32),
            pltpu.SemaphoreType.DMA,
        ],
        compiler_params=pltpu.CompilerParams(use_tc_tiling_on_sc=False),
    )
    def prop_kernel(xz_hbm, src_hbm, dst_hbm, out_hbm,
                    idx_s, idx_d, idx_st, idx_dt, rows, accum, sem):
        c = lax.axis_index("c")
        s = lax.axis_index("s")
        wid = s * NC + c
        zeros16 = jnp.zeros((LANES,), jnp.float32)

        def zrow(r, carry):
            for j in range(F // LANES):
                rows[r, pl.ds(j * LANES, LANES)] = zeros16
            return carry

        lax.fori_loop(0, ZCH, zrow, None)
        for k in range(RPA // ZCH):
            pltpu.sync_copy(rows.at[pl.ds(0, ZCH)],
                            accum.at[pl.ds(s * RPA + k * ZCH, ZCH)])

        @pl.when(s == 0)
        def _ztail():
            pltpu.sync_copy(rows.at[pl.ds(0, REM)],
                            accum.at[pl.ds(NS * RPA, REM)])

        plsc.subcore_barrier()

        ebase = wid * EPT

        def chunk(ci, carry):
            off = ebase + ci * CH
            pltpu.sync_copy(src_hbm.at[pl.ds(off, CH)], idx_s)
            pltpu.sync_copy(dst_hbm.at[pl.ds(off, CH)], idx_d)
            pltpu.async_copy(xz_hbm.at[idx_s], rows, sem).wait()
            pltpu.sync_copy(rows, accum.at[idx_d], add=True)
            return carry

        lax.fori_loop(0, NFULL, chunk, None)
        toff = ebase + NFULL * CH
        pltpu.sync_copy(src_hbm.at[pl.ds(toff, TAIL)], idx_st)
        pltpu.sync_copy(dst_hbm.at[pl.ds(toff, TAIL)], idx_dt)
        pltpu.async_copy(xz_hbm.at[idx_st], rows.at[pl.ds(0, TAIL)], sem).wait()
        pltpu.sync_copy(rows.at[pl.ds(0, TAIL)], accum.at[idx_dt], add=True)

        plsc.subcore_barrier()
        pltpu.sync_copy(accum.at[pl.ds(s * RPA, RPA)],
                        out_hbm.at[c, pl.ds(s * RPA, RPA)])

        @pl.when(s == 0)
        def _rtail():
            pltpu.sync_copy(accum.at[pl.ds(NS * RPA, REM)],
                            out_hbm.at[c, pl.ds(NS * RPA, REM)])

    return prop_kernel


_BR = 1000
_GRID = N // _BR


def _mm1_body(x_ref, degt_ref, w1_ref, xz_ref, dis_ref):
    dsum = (jnp.sum(degt_ref[0], axis=1, keepdims=True)
            + jnp.sum(degt_ref[1], axis=1, keepdims=True))
    deg = 1.0 + dsum / float(LANES)
    dis = lax.rsqrt(deg)
    xw = jnp.dot(x_ref[...], w1_ref[...],
                 preferred_element_type=jnp.float32, precision=_HIGH)
    xz_ref[...] = xw * dis
    dis_ref[...] = dis


def _l_body(part_ref, xz_ref, dis_ref, b_ref, a_ref, st_ref):
    i = pl.program_id(0)
    acc = part_ref[0] + part_ref[1] + xz_ref[...]
    pre = acc * dis_ref[...] + b_ref[...][None, :]
    a = jnp.maximum(pre, 0.0)
    a_ref[...] = a
    st = jnp.stack([jnp.sum(a, axis=0), jnp.sum(a * a, axis=0)], axis=0)

    @pl.when(i == 0)
    def _init():
        st_ref[...] = st

    @pl.when(i != 0)
    def _acc():
        st_ref[...] = st_ref[...] + st


def _mm2_body(a_ref, st_ref, g_ref, be_ref, w2_ref, dis_ref, xz2_ref):
    mu = st_ref[0] / float(N)
    var = st_ref[1] / float(N) - mu * mu
    scale = g_ref[...] * lax.rsqrt(var + EPS)
    shift = be_ref[...] - mu * scale
    h = a_ref[...] * scale[None, :] + shift[None, :]
    xw2 = jnp.dot(h, w2_ref[...],
                  preferred_element_type=jnp.float32, precision=_HIGH)
    xz2_ref[...] = xw2 * dis_ref[...]


def _l2_body(part_ref, xz_ref, dis_ref, b_ref, batch_ref,
             st_ref, q_ref, cnt_ref):
    i = pl.program_id(0)
    acc = part_ref[0] + part_ref[1] + xz_ref[...]
    pre = acc * dis_ref[...] + b_ref[...][None, :]
    a = jnp.maximum(pre, 0.0)
    st = jnp.stack([jnp.sum(a, axis=0), jnp.sum(a * a, axis=0)], axis=0)
    gids = lax.broadcasted_iota(jnp.int32, (_BR, G), 1)
    oh = (batch_ref[...] == gids).astype(jnp.float32)
    q = lax.dot_general(oh, a, (((0,), (0,)), ((), ())),
                        preferred_element_type=jnp.float32, precision=_HIGH)
    cnt = jnp.sum(oh, axis=0)

    @pl.when(i == 0)
    def _init():
        st_ref[...] = st
        q_ref[...] = q
        cnt_ref[...] = cnt

    @pl.when(i != 0)
    def _acc():
        st_ref[...] = st_ref[...] + st
        q_ref[...] = q_ref[...] + q
        cnt_ref[...] = cnt_ref[...] + cnt


def _head_body(q_ref, cnt_ref, st_ref, g2_ref, be2_ref,
               wf1_ref, bf1_ref, g3_ref, be3_ref,
               wf2_ref, bf2_ref, wf3_ref, bf3_ref, out_ref):
    mu = st_ref[0] / float(N)
    var = st_ref[1] / float(N) - mu * mu
    scale = g2_ref[...] * lax.rsqrt(var + EPS)
    shift = be2_ref[...] - mu * scale
    p = q_ref[...] * scale[None, :] + cnt_ref[...][:, None] * shift[None, :]
    p = jnp.maximum(jnp.dot(p, wf1_ref[...],
                            preferred_element_type=jnp.float32,
                            precision=_HIGH) + bf1_ref[...][None, :], 0.0)
    mu3 = jnp.mean(p, axis=0)
    var3 = jnp.mean((p - mu3[None, :]) ** 2, axis=0)
    p = g3_ref[...][None, :] * (p - mu3[None, :]) * lax.rsqrt(var3 + EPS) \
        + be3_ref[...][None, :]
    p = jnp.maximum(jnp.dot(p, wf2_ref[...],
                            preferred_element_type=jnp.float32,
                            precision=_HIGH) + bf2_ref[...][None, :], 0.0)
    z = jnp.dot(p, wf3_ref[...],
                preferred_element_type=jnp.float32,
                precision=_HIGH) + bf3_ref[...][None, :]
    m = jnp.max(z, axis=1, keepdims=True)
    out_ref[...] = z - m - jnp.log(jnp.sum(jnp.exp(z - m), axis=1,
                                           keepdims=True))


def _full(shape):
    return pl.BlockSpec(shape, lambda i: tuple(0 for _ in shape))


def _rows(bshape):
    return pl.BlockSpec(bshape, lambda i: (i,) + tuple(0 for _ in bshape[1:]))


def _tc_mm1(x, degt, w1):
    return pl.pallas_call(
        _mm1_body,
        grid=(_GRID,),
        in_specs=[
            _rows((_BR, D)),
            pl.BlockSpec((NC, _BR, LANES), lambda i: (0, i, 0)),
            _full((D, 128)),
        ],
        out_specs=[_rows((_BR, 128)), _rows((_BR, 1))],
        out_shape=[jax.ShapeDtypeStruct((N, 128), jnp.float32),
                   jax.ShapeDtypeStruct((N, 1), jnp.float32)],
    )(x, degt, w1)


def _tc_layer(part, xz, dis, b, F):
    return pl.pallas_call(
        _l_body,
        grid=(_GRID,),
        in_specs=[
            pl.BlockSpec((NC, _BR, F), lambda i: (0, i, 0)),
            _rows((_BR, F)),
            _rows((_BR, 1)),
            _full((F,)),
        ],
        out_specs=[_rows((_BR, F)), _full((2, F))],
        out_shape=[jax.ShapeDtypeStruct((N, F), jnp.float32),
                   jax.ShapeDtypeStruct((2, F), jnp.float32)],
    )(part, xz, dis, b)


def _tc_mm2(a1, st1, g1, be1, w2, dis):
    return pl.pallas_call(
        _mm2_body,
        grid=(_GRID,),
        in_specs=[
            _rows((_BR, 128)),
            _full((2, 128)),
            _full((128,)),
            _full((128,)),
            _full((128, 64)),
            _rows((_BR, 1)),
        ],
        out_specs=_rows((_BR, 64)),
        out_shape=jax.ShapeDtypeStruct((N, 64), jnp.float32),
    )(a1, st1, g1, be1, w2, dis)


def _tc_l2(part, xz2, dis, b2, batch):
    return pl.pallas_call(
        _l2_body,
        grid=(_GRID,),
        in_specs=[
            pl.BlockSpec((NC, _BR, 64), lambda i: (0, i, 0)),
            _rows((_BR, 64)),
            _rows((_BR, 1)),
            _full((64,)),
            _rows((_BR, 1)),
        ],
        out_specs=[_full((2, 64)), _full((G, 64)), _full((G,))],
        out_shape=[jax.ShapeDtypeStruct((2, 64), jnp.float32),
                   jax.ShapeDtypeStruct((G, 64), jnp.float32),
                   jax.ShapeDtypeStruct((G,), jnp.float32)],
    )(part, xz2, dis, b2, batch)


def _tc_head(q, cnt, st2, g2, be2, wf1, bf1, g3, be3, wf2, bf2, wf3, bf3):
    return pl.pallas_call(
        _head_body,
        out_shape=jax.ShapeDtypeStruct((G, 2), jnp.float32),
    )(q, cnt, st2, g2, be2, wf1, bf1, g3, be3, wf2, bf2, wf3, bf3)


def kernel(x, edge_index, batch, W1, b1, g1, be1, W2, b2, g2, be2,
           Wf1, bf1, g3, be3, Wf2, bf2, Wf3, bf3):
    src = edge_index[0]
    dst = edge_index[1]
    degt = _make_deg()(dst)
    xz1, dis = _tc_mm1(x, degt, W1)
    part1 = _make_prop(128)(xz1, src, dst)
    a1, st1 = _tc_layer(part1, xz1, dis, b1, 128)
    xz2 = _tc_mm2(a1, st1, g1, be1, W2, dis)
    part2 = _make_prop(64)(xz2, src, dst)
    st2, q, cnt = _tc_l2(part2, xz2, dis, b2, batch[:, None])
    return _tc_head(q, cnt, st2, g2, be2,
                    Wf1, bf1, g3, be3, Wf2, bf2, Wf3, bf3)
